# Initial kernel scaffold; baseline (speedup 1.0000x reference)
import jax
import jax.numpy as jnp
from jax.experimental import pallas as pl


def kernel(z_e, codebook0, codebook1, codebook2):
    raise NotImplementedError("write your pallas kernel here")


# speedup vs baseline: 1.1003x; 1.1003x over previous
import functools

import jax
import jax.numpy as jnp
from jax.experimental import pallas as pl
from jax.experimental.pallas import tpu as pltpu

EMBED_DIM = 256
NUM_CODES = 1024
COMMITMENT_COSTS = (0.25, 0.5, 1.0)
BLK = 1024


def _hrvq_kernel(z_ref, cb0_ref, cb1_ref, cb2_ref,
                 zq_ref, i0_ref, i1_ref, i2_ref, loss_ref):
    step = pl.program_id(0)
    nsteps = pl.num_programs(0)

    @pl.when(step == 0)
    def _init():
        loss_ref[0, 0] = jnp.float32(0.0)

    r = z_ref[...]
    zq = jnp.zeros_like(r)
    loss = jnp.float32(0.0)
    idx_refs = (i0_ref, i1_ref, i2_ref)
    cb_refs = (cb0_ref, cb1_ref, cb2_ref)

    for lvl in range(3):
        cb = cb_refs[lvl][...]
        cbsq = jnp.sum(cb * cb, axis=1)
        rsq = jnp.sum(r * r, axis=1, keepdims=True)
        prod = jax.lax.dot_general(
            r, cb, (((1,), (1,)), ((), ())),
            preferred_element_type=jnp.float32)
        dist = rsq - 2.0 * prod + cbsq[None, :]
        mind = jnp.min(dist, axis=1, keepdims=True)
        lane = jax.lax.broadcasted_iota(jnp.int32, dist.shape, 1)
        idx = jnp.min(jnp.where(dist == mind, lane, NUM_CODES),
                      axis=1)
        idx_refs[lvl][0, 0, :] = idx
        onehot = (lane == idx[:, None]).astype(jnp.float32)
        q = jax.lax.dot_general(
            onehot, cb, (((1,), (0,)), ((), ())),
            preferred_element_type=jnp.float32,
            precision=jax.lax.Precision.HIGHEST)
        zq = zq + q
        r = r - q
        loss = loss + (1.0 + COMMITMENT_COSTS[lvl]) * jnp.sum(r * r)

    zq_ref[...] = zq
    loss_ref[0, 0] += loss

    @pl.when(step == nsteps - 1)
    def _scale():
        loss_ref[0, 0] = loss_ref[0, 0] / jnp.float32(16 * 1024 * EMBED_DIM)


@jax.jit
def kernel(z_e, codebook0, codebook1, codebook2):
    B, N, D = z_e.shape
    BN = B * N
    nb = BN // BLK
    flat = z_e.reshape(BN, D)

    cb_spec = pl.BlockSpec((NUM_CODES, D), lambda i: (0, 0))
    idx_spec = pl.BlockSpec((1, 1, BLK), lambda i: (i, 0, 0))

    zq, i0, i1, i2, loss = pl.pallas_call(
        _hrvq_kernel,
        grid=(nb,),
        in_specs=[pl.BlockSpec((BLK, D), lambda i: (i, 0)),
                  cb_spec, cb_spec, cb_spec],
        out_specs=[pl.BlockSpec((BLK, D), lambda i: (i, 0)),
                   idx_spec, idx_spec, idx_spec,
                   pl.BlockSpec(memory_space=pltpu.SMEM)],
        out_shape=[jax.ShapeDtypeStruct((BN, D), jnp.float32),
                   jax.ShapeDtypeStruct((nb, 1, BLK), jnp.int32),
                   jax.ShapeDtypeStruct((nb, 1, BLK), jnp.int32),
                   jax.ShapeDtypeStruct((nb, 1, BLK), jnp.int32),
                   jax.ShapeDtypeStruct((1, 1), jnp.float32)],
        compiler_params=pltpu.CompilerParams(
            dimension_semantics=("arbitrary",)),
    )(flat, codebook0, codebook1, codebook2)

    z_q = zq.reshape(B, N, D)
    idx0 = i0.reshape(B, N)
    idx1 = i1.reshape(B, N)
    idx2 = i2.reshape(B, N)
    return z_q, loss[0, 0], idx0, idx1, idx2


# speedup vs baseline: 1.6533x; 1.5025x over previous
import functools

import jax
import jax.numpy as jnp
from jax import lax
from jax.experimental import pallas as pl
from jax.experimental.pallas import tpu as pltpu
from jax.experimental.pallas import tpu_sc as plsc

EMBED_DIM = 256
NUM_CODES = 1024
BN = 16384
BLK = 1024

_DN_T = (((1,), (1,)), ((), ()))

_info = plsc.get_sparse_core_info()
_NC, _NS = _info.num_cores, _info.num_subcores
_NW = _NC * _NS
_B_PER_W = BN // _NW
_CHUNK = 256


def _sc_gather(table_hbm, idx_hbm, out_hbm, idx_v, rows_v, sem):
    wid = lax.axis_index("s") * _NC + lax.axis_index("c")
    base = wid * _B_PER_W
    pltpu.sync_copy(idx_hbm.at[pl.ds(base, _B_PER_W)], idx_v)
    for ch in range(_B_PER_W // _CHUNK):
        pltpu.async_copy(
            table_hbm.at[idx_v.at[pl.ds(ch * _CHUNK, _CHUNK)]], rows_v, sem
        ).wait()
        pltpu.sync_copy(rows_v, out_hbm.at[pl.ds(base + ch * _CHUNK, _CHUNK)])


def _gather_rows(table, idx_flat):
    mesh = plsc.VectorSubcoreMesh(core_axis_name="c", subcore_axis_name="s")
    k = functools.partial(
        pl.kernel, mesh=mesh,
        out_type=jax.ShapeDtypeStruct((BN, EMBED_DIM), jnp.float32),
        scratch_types=[pltpu.VMEM((_B_PER_W,), jnp.int32),
                       pltpu.VMEM((_CHUNK, EMBED_DIM), jnp.float32),
                       pltpu.SemaphoreType.DMA],
    )(_sc_gather)
    return k(table, idx_flat)


def _dist_argmin(r, cb):
    cbsq = jnp.sum(cb * cb, axis=1)
    rsq = jnp.sum(r * r, axis=1, keepdims=True)
    prod = jax.lax.dot_general(r, cb, _DN_T,
                               preferred_element_type=jnp.float32)
    dist = rsq - 2.0 * prod + cbsq[None, :]
    mind = jnp.min(dist, axis=1, keepdims=True)
    lane = jax.lax.broadcasted_iota(jnp.int32, dist.shape, 1)
    return jnp.min(jnp.where(dist == mind, lane, NUM_CODES), axis=1)


def _level0_body(r_ref, cb_ref, idx_ref):
    idx_ref[0, 0, :] = _dist_argmin(r_ref[...], cb_ref[...])


def _levelN_body(rp_ref, qp_ref, cb_ref, idx_ref, rn_ref, ls_ref):
    @pl.when(pl.program_id(0) == 0)
    def _():
        ls_ref[0, 0] = jnp.float32(0.0)
    r = rp_ref[...] - qp_ref[...]
    rn_ref[...] = r
    idx_ref[0, 0, :] = _dist_argmin(r, cb_ref[...])
    ls_ref[0, 0] += jnp.sum(r * r)


def _final_body(z_ref, rp_ref, qp_ref, zq_ref, ls_ref):
    @pl.when(pl.program_id(0) == 0)
    def _():
        ls_ref[0, 0] = jnp.float32(0.0)
    r = rp_ref[...] - qp_ref[...]
    zq_ref[...] = z_ref[...] - r
    ls_ref[0, 0] += jnp.sum(r * r)


_rspec = pl.BlockSpec((BLK, EMBED_DIM), lambda i: (i, 0))
_cspec = pl.BlockSpec((NUM_CODES, EMBED_DIM), lambda i: (0, 0))
_ispec = pl.BlockSpec((1, 1, BLK), lambda i: (i, 0, 0))
_ishape = jax.ShapeDtypeStruct((BN // BLK, 1, BLK), jnp.int32)
_sspec = pl.BlockSpec(memory_space=pltpu.SMEM)
_sshape = jax.ShapeDtypeStruct((1, 1), jnp.float32)
_fshape = jax.ShapeDtypeStruct((BN, EMBED_DIM), jnp.float32)
_arb = pltpu.CompilerParams(dimension_semantics=("arbitrary",))


@jax.jit
def kernel(z_e, codebook0, codebook1, codebook2):
    B, N, D = z_e.shape
    flat = z_e.reshape(BN, D)
    nb = BN // BLK

    i0 = pl.pallas_call(_level0_body, grid=(nb,),
                        in_specs=[_rspec, _cspec], out_specs=_ispec,
                        out_shape=_ishape, compiler_params=_arb)(flat, codebook0)
    q0 = _gather_rows(codebook0, i0.reshape(BN))

    i1, r1, s0 = pl.pallas_call(_levelN_body, grid=(nb,),
                                in_specs=[_rspec, _rspec, _cspec],
                                out_specs=[_ispec, _rspec, _sspec],
                                out_shape=[_ishape, _fshape, _sshape],
                                compiler_params=_arb)(flat, q0, codebook1)
    q1 = _gather_rows(codebook1, i1.reshape(BN))

    i2, r2, s1 = pl.pallas_call(_levelN_body, grid=(nb,),
                                in_specs=[_rspec, _rspec, _cspec],
                                out_specs=[_ispec, _rspec, _sspec],
                                out_shape=[_ishape, _fshape, _sshape],
                                compiler_params=_arb)(r1, q1, codebook2)
    q2 = _gather_rows(codebook2, i2.reshape(BN))

    zq, s2 = pl.pallas_call(_final_body, grid=(nb,),
                            in_specs=[_rspec, _rspec, _rspec],
                            out_specs=[_rspec, _sspec],
                            out_shape=[_fshape, _sshape],
                            compiler_params=_arb)(flat, r2, q2)

    scale = jnp.float32(1.0 / (BN * D))
    loss = (1.25 * s0[0, 0] + 1.5 * s1[0, 0] + 2.0 * s2[0, 0]) * scale
    return (zq.reshape(B, N, D), loss,
            i0.reshape(B, N), i1.reshape(B, N), i2.reshape(B, N))


# speedup vs baseline: 1.8269x; 1.1050x over previous
import functools

import jax
import jax.numpy as jnp
from jax import lax
from jax.experimental import pallas as pl
from jax.experimental.pallas import tpu as pltpu
from jax.experimental.pallas import tpu_sc as plsc

EMBED_DIM = 256
NUM_CODES = 1024
BN = 16384
BLK = 1024

_DN_T = (((1,), (1,)), ((), ()))

_info = plsc.get_sparse_core_info()
_NC, _NS = _info.num_cores, _info.num_subcores
_NW = _NC * _NS
_B_PER_W = BN // _NW
_CHUNK = 256


def _sc_gather(table_hbm, idx_hbm, out_hbm, idx_v, rows_v, sem):
    wid = lax.axis_index("s") * _NC + lax.axis_index("c")
    base = wid * _B_PER_W
    pltpu.sync_copy(idx_hbm.at[pl.ds(base, _B_PER_W)], idx_v)
    for ch in range(_B_PER_W // _CHUNK):
        pltpu.async_copy(
            table_hbm.at[idx_v.at[pl.ds(ch * _CHUNK, _CHUNK)]], rows_v, sem
        ).wait()
        pltpu.sync_copy(rows_v, out_hbm.at[pl.ds(base + ch * _CHUNK, _CHUNK)])


def _gather_rows(table, idx_flat):
    mesh = plsc.VectorSubcoreMesh(core_axis_name="c", subcore_axis_name="s")
    k = functools.partial(
        pl.kernel, mesh=mesh,
        out_type=jax.ShapeDtypeStruct((BN, EMBED_DIM), jnp.float32),
        scratch_types=[pltpu.VMEM((_B_PER_W,), jnp.int32),
                       pltpu.VMEM((_CHUNK, EMBED_DIM), jnp.float32),
                       pltpu.SemaphoreType.DMA],
    )(_sc_gather)
    return k(table, idx_flat)


def _dist_argmin(r, cb):
    cbsq = jnp.sum(cb * cb, axis=1)
    rsq = jnp.sum(r * r, axis=1, keepdims=True)
    prod = jax.lax.dot_general(r, cb, _DN_T,
                               preferred_element_type=jnp.float32)
    dist = rsq - 2.0 * prod + cbsq[None, :]
    mind = jnp.min(dist, axis=1, keepdims=True)
    lane = jax.lax.broadcasted_iota(
        jnp.int32, (1, NUM_CODES), 1).astype(jnp.float32)
    idxf = jnp.min(jnp.where(dist == mind, lane, jnp.float32(NUM_CODES)),
                   axis=1)
    return idxf.astype(jnp.int32)


def _level0_body(r_ref, cb_ref, idx_ref):
    idx_ref[0, 0, :] = _dist_argmin(r_ref[...], cb_ref[...])


def _levelN_body(rp_ref, qp_ref, cb_ref, idx_ref, rn_ref, ls_ref):
    @pl.when(pl.program_id(0) == 0)
    def _():
        ls_ref[0, 0] = jnp.float32(0.0)
    r = rp_ref[...] - qp_ref[...]
    rn_ref[...] = r
    idx_ref[0, 0, :] = _dist_argmin(r, cb_ref[...])
    ls_ref[0, 0] += jnp.sum(r * r)


def _final_body(z_ref, rp_ref, qp_ref, zq_ref, ls_ref):
    @pl.when(pl.program_id(0) == 0)
    def _():
        ls_ref[0, 0] = jnp.float32(0.0)
    r = rp_ref[...] - qp_ref[...]
    zq_ref[...] = z_ref[...] - r
    ls_ref[0, 0] += jnp.sum(r * r)


_rspec = pl.BlockSpec((BLK, EMBED_DIM), lambda i: (i, 0))
_cspec = pl.BlockSpec((NUM_CODES, EMBED_DIM), lambda i: (0, 0))
_ispec = pl.BlockSpec((1, 1, BLK), lambda i: (i, 0, 0))
_ishape = jax.ShapeDtypeStruct((BN // BLK, 1, BLK), jnp.int32)
_sspec = pl.BlockSpec(memory_space=pltpu.SMEM)
_sshape = jax.ShapeDtypeStruct((1, 1), jnp.float32)
_fshape = jax.ShapeDtypeStruct((BN, EMBED_DIM), jnp.float32)
_arb = pltpu.CompilerParams(dimension_semantics=("arbitrary",))


@jax.jit
def kernel(z_e, codebook0, codebook1, codebook2):
    B, N, D = z_e.shape
    flat = z_e.reshape(BN, D)
    nb = BN // BLK

    i0 = pl.pallas_call(_level0_body, grid=(nb,),
                        in_specs=[_rspec, _cspec], out_specs=_ispec,
                        out_shape=_ishape, compiler_params=_arb)(flat, codebook0)
    q0 = _gather_rows(codebook0, i0.reshape(BN))

    i1, r1, s0 = pl.pallas_call(_levelN_body, grid=(nb,),
                                in_specs=[_rspec, _rspec, _cspec],
                                out_specs=[_ispec, _rspec, _sspec],
                                out_shape=[_ishape, _fshape, _sshape],
                                compiler_params=_arb)(flat, q0, codebook1)
    q1 = _gather_rows(codebook1, i1.reshape(BN))

    i2, r2, s1 = pl.pallas_call(_levelN_body, grid=(nb,),
                                in_specs=[_rspec, _rspec, _cspec],
                                out_specs=[_ispec, _rspec, _sspec],
                                out_shape=[_ishape, _fshape, _sshape],
                                compiler_params=_arb)(r1, q1, codebook2)
    q2 = _gather_rows(codebook2, i2.reshape(BN))

    zq, s2 = pl.pallas_call(_final_body, grid=(nb,),
                            in_specs=[_rspec, _rspec, _rspec],
                            out_specs=[_rspec, _sspec],
                            out_shape=[_fshape, _sshape],
                            compiler_params=_arb)(flat, r2, q2)

    scale = jnp.float32(1.0 / (BN * D))
    loss = (1.25 * s0[0, 0] + 1.5 * s1[0, 0] + 2.0 * s2[0, 0]) * scale
    return (zq.reshape(B, N, D), loss,
            i0.reshape(B, N), i1.reshape(B, N), i2.reshape(B, N))
